# baseline (speedup 1.0000x reference)
import jax
import jax.numpy as jnp
from jax.experimental import pallas as pl
from jax.experimental.pallas import tpu as pltpu

BATCH = 4096
IN_DIM = 784
MID = 1024
CODE = 8192
SD = 64
NS = 128
KN = 410
KS = 16
CC = 512

BT_A = 512
BT_B = 512
CT_B = 2048
KB = 1024
BT_D = 128


def _enc1_kernel(x_ref, w_ref, b_ref, o_ref):
    o_ref[...] = jnp.maximum(
        jnp.dot(x_ref[...], w_ref[...], preferred_element_type=jnp.float32)
        + b_ref[...], 0.0)


def _colsum_kernel(h1_ref, w2_ref, b2_ref, cs_ref):
    i = pl.program_id(1)
    h2 = jnp.maximum(
        jnp.dot(h1_ref[...], w2_ref[...], preferred_element_type=jnp.float32)
        + b2_ref[...], 0.0)
    ps = jnp.sum(h2, axis=0, keepdims=True)

    @pl.when(i == 0)
    def _():
        cs_ref[...] = ps

    @pl.when(i != 0)
    def _():
        cs_ref[...] = cs_ref[...] + ps


def _nmask_kernel(cs_ref, m_ref, slot_ref):
    bits = jax.lax.bitcast_convert_type(cs_ref[...], jnp.int32)

    def tstep(k, t):
        cand = t | (1 << (30 - k))
        cnt = jnp.sum((bits >= cand).astype(jnp.int32))
        return jnp.where(cnt >= KN, cand, t)

    t = jax.lax.fori_loop(0, 31, tstep, jnp.int32(0))
    m = jnp.sum((bits > t).astype(jnp.int32))
    r = KN - m
    tie = bits == t
    idx = jax.lax.broadcasted_iota(jnp.int32, (1, CODE), 1)

    def jstep(k, J):
        cand = J | (1 << (13 - k))
        g = jnp.sum((tie & (idx < cand)).astype(jnp.int32))
        return jnp.where((cand <= CODE) & (g <= r), cand, J)

    J = jax.lax.fori_loop(0, 14, jstep, jnp.int32(0))
    mask = (bits > t) | (tie & (idx < J))
    m_ref[...] = jnp.where(mask, 1.0, 0.0)
    x = m_ref[...]
    sh = 1
    while sh < CODE:
        x = x + jnp.concatenate(
            [jnp.zeros((1, sh), jnp.float32), x[:, :CODE - sh]], axis=1)
        sh *= 2
    slot_ref[...] = jnp.where(mask, x - 1.0, -1.0)


def _build_kernel(slot_ref, w3_ref, pt_ref, w3c_ref, st_ref, sel_ref):
    k = pl.program_id(0)
    slot_blk = slot_ref[...]
    jrow = jax.lax.broadcasted_iota(jnp.int32, (CC, KB), 0)
    pt_blk = (jrow == slot_blk.astype(jnp.int32)).astype(jnp.float32)
    pt_blk = jnp.where(slot_blk >= 0.0, pt_blk, 0.0)
    pt_ref[...] = pt_blk.astype(jnp.bfloat16)
    stripe = ((jax.lax.broadcasted_iota(jnp.int32, (1, KB), 1)
               + k * KB) // SD).astype(jnp.bfloat16)
    st_part = jax.lax.dot_general(
        stripe, pt_blk.astype(jnp.bfloat16), (((1,), (1,)), ((), ())),
        preferred_element_type=jnp.float32)
    w3c_part = jnp.dot(pt_blk.astype(jnp.bfloat16),
                       w3_ref[...].astype(jnp.bfloat16),
                       preferred_element_type=jnp.float32)

    @pl.when(k == 0)
    def _():
        st_ref[...] = st_part
        w3c_ref[...] = w3c_part

    @pl.when(k != 0)
    def _():
        st_ref[...] = st_ref[...] + st_part
        w3c_ref[...] = w3c_ref[...] + w3c_part

    @pl.when(k == CODE // KB - 1)
    def _():
        srow = jax.lax.broadcasted_iota(jnp.int32, (NS, CC), 0)
        sel_ref[...] = (srow == st_ref[...].astype(jnp.int32)).astype(
            jnp.bfloat16)


def _dec_kernel(h1_ref, w2_ref, b2_ref, nm_ref, pt_ref, w3c_ref, sel_ref,
                b3_ref, w4_ref, b4_ref, o_ref):
    h2 = jnp.maximum(
        jnp.dot(h1_ref[...], w2_ref[...], preferred_element_type=jnp.float32)
        + b2_ref[...], 0.0)
    hm = h2 * nm_ref[...]
    hm_hi = hm.astype(jnp.bfloat16)
    hm_lo = (hm - hm_hi.astype(jnp.float32)).astype(jnp.bfloat16)
    sn = jax.lax.broadcasted_iota(jnp.int32, (CODE, NS), 0) // SD
    sc = jax.lax.broadcasted_iota(jnp.int32, (CODE, NS), 1)
    S = (sn == sc).astype(jnp.bfloat16)
    ss = (jnp.dot(hm_hi, S, preferred_element_type=jnp.float32)
          + jnp.dot(hm_lo, S, preferred_element_type=jnp.float32))

    bits = jax.lax.bitcast_convert_type(ss, jnp.int32)

    def tstep(k, t):
        cand = t | (1 << (30 - k))
        cnt = jnp.sum((bits >= cand).astype(jnp.int32), axis=1, keepdims=True)
        return jnp.where(cnt >= KS, cand, t)

    t = jax.lax.fori_loop(0, 31, tstep, jnp.zeros((BT_D, 1), jnp.int32))
    m = jnp.sum((bits > t).astype(jnp.int32), axis=1, keepdims=True)
    r = KS - m
    tie = bits == t
    sidx = jax.lax.broadcasted_iota(jnp.int32, (1, NS), 1)

    def jstep(k, J):
        cand = J | (1 << (7 - k))
        g = jnp.sum((tie & (sidx < cand)).astype(jnp.int32), axis=1,
                    keepdims=True)
        return jnp.where((cand <= NS) & (g <= r), cand, J)

    J = jax.lax.fori_loop(0, 8, jstep, jnp.zeros((BT_D, 1), jnp.int32))
    smask = ((bits > t) | (tie & (sidx < J))).astype(jnp.bfloat16)

    c = jax.lax.dot_general(hm_hi, pt_ref[...], (((1,), (1,)), ((), ())),
                            preferred_element_type=jnp.float32)
    smask_c = jnp.dot(smask, sel_ref[...],
                      preferred_element_type=jnp.float32)
    cm = (c * smask_c).astype(jnp.bfloat16)
    h3 = jnp.maximum(
        jnp.dot(cm, w3c_ref[...].astype(jnp.bfloat16),
                preferred_element_type=jnp.float32) + b3_ref[...], 0.0)
    o_ref[...] = jnp.maximum(
        jnp.dot(h3.astype(jnp.bfloat16), w4_ref[...],
                preferred_element_type=jnp.float32) + b4_ref[...], 0.0)


@jax.jit
def kernel(x, W1, b1, W2, b2, W3, b3, W4, b4):
    b1r = b1.reshape(1, MID)
    b2r = b2.reshape(1, CODE)
    b3r = b3.reshape(1, MID)
    b4r = b4.reshape(1, IN_DIM)

    h1 = pl.pallas_call(
        _enc1_kernel,
        grid=(BATCH // BT_A,),
        in_specs=[
            pl.BlockSpec((BT_A, IN_DIM), lambda i: (i, 0)),
            pl.BlockSpec((IN_DIM, MID), lambda i: (0, 0)),
            pl.BlockSpec((1, MID), lambda i: (0, 0)),
        ],
        out_specs=pl.BlockSpec((BT_A, MID), lambda i: (i, 0)),
        out_shape=jax.ShapeDtypeStruct((BATCH, MID), jnp.float32),
    )(x, W1, b1r)

    cs = pl.pallas_call(
        _colsum_kernel,
        grid=(CODE // CT_B, BATCH // BT_B),
        in_specs=[
            pl.BlockSpec((BT_B, MID), lambda j, i: (i, 0)),
            pl.BlockSpec((MID, CT_B), lambda j, i: (0, j)),
            pl.BlockSpec((1, CT_B), lambda j, i: (0, j)),
        ],
        out_specs=pl.BlockSpec((1, CT_B), lambda j, i: (0, j)),
        out_shape=jax.ShapeDtypeStruct((1, CODE), jnp.float32),
    )(h1, W2, b2r)

    nmask, slotm = pl.pallas_call(
        _nmask_kernel,
        out_shape=[
            jax.ShapeDtypeStruct((1, CODE), jnp.float32),
            jax.ShapeDtypeStruct((1, CODE), jnp.float32),
        ],
    )(cs)

    pt, w3c, st, sel = pl.pallas_call(
        _build_kernel,
        grid=(CODE // KB,),
        in_specs=[
            pl.BlockSpec((1, KB), lambda k: (0, k)),
            pl.BlockSpec((KB, MID), lambda k: (k, 0)),
        ],
        out_specs=[
            pl.BlockSpec((CC, KB), lambda k: (0, k)),
            pl.BlockSpec((CC, MID), lambda k: (0, 0)),
            pl.BlockSpec((1, CC), lambda k: (0, 0)),
            pl.BlockSpec((NS, CC), lambda k: (0, 0)),
        ],
        out_shape=[
            jax.ShapeDtypeStruct((CC, CODE), jnp.bfloat16),
            jax.ShapeDtypeStruct((CC, MID), jnp.float32),
            jax.ShapeDtypeStruct((1, CC), jnp.float32),
            jax.ShapeDtypeStruct((NS, CC), jnp.bfloat16),
        ],
    )(slotm, W3)

    out = pl.pallas_call(
        _dec_kernel,
        grid=(BATCH // BT_D,),
        in_specs=[
            pl.BlockSpec((BT_D, MID), lambda i: (i, 0)),
            pl.BlockSpec((MID, CODE), lambda i: (0, 0)),
            pl.BlockSpec((1, CODE), lambda i: (0, 0)),
            pl.BlockSpec((1, CODE), lambda i: (0, 0)),
            pl.BlockSpec((CC, CODE), lambda i: (0, 0)),
            pl.BlockSpec((CC, MID), lambda i: (0, 0)),
            pl.BlockSpec((NS, CC), lambda i: (0, 0)),
            pl.BlockSpec((1, MID), lambda i: (0, 0)),
            pl.BlockSpec((MID, IN_DIM), lambda i: (0, 0)),
            pl.BlockSpec((1, IN_DIM), lambda i: (0, 0)),
        ],
        out_specs=pl.BlockSpec((BT_D, IN_DIM), lambda i: (i, 0)),
        out_shape=jax.ShapeDtypeStruct((BATCH, IN_DIM), jnp.float32),
        compiler_params=pltpu.CompilerParams(
            vmem_limit_bytes=100 * 1024 * 1024,
        ),
    )(h1, W2, b2r, nmask, pt, w3c, sel, b3r, W4.astype(jnp.bfloat16), b4r)
    return out


# speedup vs baseline: 2.2147x; 2.2147x over previous
import functools

import jax
import jax.numpy as jnp
from jax import lax
from jax.experimental import pallas as pl
from jax.experimental.pallas import tpu as pltpu
from jax.experimental.pallas import tpu_sc as plsc

BATCH = 4096
IN_DIM = 784
MID = 1024
CODE = 8192
SD = 64
NS = 128
KN = 410
KS = 16
CC = 512

BT_A = 512
BT_B = 512
CT_B = 2048
KB = 1024
BT_D = 512
NW = 32
RPW = CC // NW


def _enc1_kernel(x_ref, w_ref, b_ref, o_ref):
    o_ref[...] = jnp.maximum(
        jnp.dot(x_ref[...], w_ref[...], preferred_element_type=jnp.float32)
        + b_ref[...], 0.0)


def _enc2_kernel(h1_ref, w2_ref, b2_ref, h2t_ref, cs_ref):
    i = pl.program_id(1)
    h2 = jnp.maximum(
        jnp.dot(h1_ref[...], w2_ref[...], preferred_element_type=jnp.float32)
        + b2_ref[...], 0.0)
    h2t_ref[...] = jnp.transpose(h2)
    ps = jnp.sum(h2, axis=0, keepdims=True)

    @pl.when(i == 0)
    def _():
        cs_ref[...] = ps

    @pl.when(i != 0)
    def _():
        cs_ref[...] = cs_ref[...] + ps


def _nmask_kernel(cs_ref, m_ref, slot_ref):
    bits = lax.bitcast_convert_type(cs_ref[...], jnp.int32)

    def tstep(k, t):
        cand = t | (1 << (30 - k))
        cnt = jnp.sum((bits >= cand).astype(jnp.int32))
        return jnp.where(cnt >= KN, cand, t)

    t = lax.fori_loop(0, 31, tstep, jnp.int32(0))
    m = jnp.sum((bits > t).astype(jnp.int32))
    r = KN - m
    tie = bits == t
    idx = lax.broadcasted_iota(jnp.int32, (1, CODE), 1)

    def jstep(k, J):
        cand = J | (1 << (13 - k))
        g = jnp.sum((tie & (idx < cand)).astype(jnp.int32))
        return jnp.where((cand <= CODE) & (g <= r), cand, J)

    J = lax.fori_loop(0, 14, jstep, jnp.int32(0))
    mask = (bits > t) | (tie & (idx < J))
    m_ref[...] = jnp.where(mask, 1.0, 0.0)
    x = m_ref[...]
    sh = 1
    while sh < CODE:
        x = x + jnp.concatenate(
            [jnp.zeros((1, sh), jnp.float32), x[:, :CODE - sh]], axis=1)
        sh *= 2
    slot_ref[...] = jnp.where(mask, x - 1.0, -1.0)


def _build_kernel(slot_ref, w3_ref, idx_ref, w3ct_ref, selp_ref, selt_ref,
                  w3c_acc, st_acc):
    k = pl.program_id(0)
    slot_blk = slot_ref[...]
    jrow = lax.broadcasted_iota(jnp.int32, (CC, KB), 0)
    pt_blk = (jrow == slot_blk.astype(jnp.int32)).astype(jnp.float32)
    pt_blk = jnp.where(slot_blk >= 0.0, pt_blk, 0.0)
    ptb = pt_blk.astype(jnp.bfloat16)
    giota = (lax.broadcasted_iota(jnp.int32, (1, KB), 1)
             + k * KB).astype(jnp.float32)
    g_hi = giota.astype(jnp.bfloat16)
    g_lo = (giota - g_hi.astype(jnp.float32)).astype(jnp.bfloat16)
    dn = (((1,), (1,)), ((), ()))
    idx_part = (lax.dot_general(g_hi, ptb, dn,
                                preferred_element_type=jnp.float32)
                + lax.dot_general(g_lo, ptb, dn,
                                  preferred_element_type=jnp.float32))
    stripe = ((lax.broadcasted_iota(jnp.int32, (1, KB), 1)
               + k * KB) // SD).astype(jnp.bfloat16)
    st_part = lax.dot_general(stripe, ptb, dn,
                              preferred_element_type=jnp.float32)
    w3c_part = jnp.dot(ptb, w3_ref[...].astype(jnp.bfloat16),
                       preferred_element_type=jnp.float32)

    @pl.when(k == 0)
    def _():
        st_acc[...] = st_part
        idx_ref[...] = idx_part
        w3c_acc[...] = w3c_part

    @pl.when(k != 0)
    def _():
        st_acc[...] = st_acc[...] + st_part
        idx_ref[...] = idx_ref[...] + idx_part
        w3c_acc[...] = w3c_acc[...] + w3c_part

    @pl.when(k == CODE // KB - 1)
    def _():
        st = st_acc[...]
        jcol = lax.broadcasted_iota(jnp.int32, (NS, CC), 1)
        srow = lax.broadcasted_iota(jnp.int32, (NS, CC), 0)
        valid = jcol < KN
        selp_ref[...] = ((srow == st.astype(jnp.int32)) & valid).astype(
            jnp.bfloat16)
        selt_ref[...] = jnp.transpose(selp_ref[...])
        w3ct_ref[...] = jnp.transpose(
            w3c_acc[...].astype(jnp.bfloat16))


def _sc_gather(h2t_hbm, idx_hbm, out_hbm, idx_v, rows_v, sem):
    wid = lax.axis_index("s") * 2 + lax.axis_index("c")
    base = wid * RPW
    pltpu.sync_copy(idx_hbm.at[pl.ds(base, RPW)], idx_v)
    pltpu.async_copy(h2t_hbm.at[idx_v], rows_v, sem).wait()
    pltpu.sync_copy(rows_v, out_hbm.at[pl.ds(base, RPW)])


def _dect_kernel(c_ref, selp_ref, selt_ref, w3ct_ref, b3c_ref, w4t_ref,
                 b4c_ref, o_ref):
    c = c_ref[...]
    c_hi = c.astype(jnp.bfloat16)
    c_lo = (c - c_hi.astype(jnp.float32)).astype(jnp.bfloat16)
    selp = selp_ref[...]
    ss = (jnp.dot(selp, c_hi, preferred_element_type=jnp.float32)
          + jnp.dot(selp, c_lo, preferred_element_type=jnp.float32))

    bits = lax.bitcast_convert_type(ss, jnp.int32)

    def tstep(k, t):
        cand = t | (1 << (30 - k))
        cnt = jnp.sum((bits >= cand).astype(jnp.int32), axis=0,
                      keepdims=True)
        return jnp.where(cnt >= KS, cand, t)

    t = lax.fori_loop(0, 31, tstep, jnp.zeros((1, BT_D), jnp.int32))
    m = jnp.sum((bits > t).astype(jnp.int32), axis=0, keepdims=True)
    r = KS - m
    tie = bits == t
    sidx = lax.broadcasted_iota(jnp.int32, (NS, 1), 0)

    def jstep(k, J):
        cand = J | (1 << (7 - k))
        g = jnp.sum((tie & (sidx < cand)).astype(jnp.int32), axis=0,
                    keepdims=True)
        return jnp.where((cand <= NS) & (g <= r), cand, J)

    J = lax.fori_loop(0, 8, jstep, jnp.zeros((1, BT_D), jnp.int32))
    smaskt = ((bits > t) | (tie & (sidx < J))).astype(jnp.bfloat16)

    em = jnp.dot(selt_ref[...], smaskt,
                 preferred_element_type=jnp.float32)
    cm = c_hi * em.astype(jnp.bfloat16)
    h3 = jnp.maximum(
        jnp.dot(w3ct_ref[...], cm, preferred_element_type=jnp.float32)
        + b3c_ref[...], 0.0)
    ot = jnp.maximum(
        jnp.dot(w4t_ref[...], h3.astype(jnp.bfloat16),
                preferred_element_type=jnp.float32) + b4c_ref[...], 0.0)
    o_ref[...] = jnp.transpose(ot)


def _w4t_kernel(w4_ref, o_ref):
    o_ref[...] = jnp.transpose(w4_ref[...].astype(jnp.bfloat16))


@jax.jit
def kernel(x, W1, b1, W2, b2, W3, b3, W4, b4):
    b1r = b1.reshape(1, MID)
    b2r = b2.reshape(1, CODE)
    b3c = b3.reshape(MID, 1)
    b4c = b4.reshape(IN_DIM, 1)

    h1 = pl.pallas_call(
        _enc1_kernel,
        grid=(BATCH // BT_A,),
        in_specs=[
            pl.BlockSpec((BT_A, IN_DIM), lambda i: (i, 0)),
            pl.BlockSpec((IN_DIM, MID), lambda i: (0, 0)),
            pl.BlockSpec((1, MID), lambda i: (0, 0)),
        ],
        out_specs=pl.BlockSpec((BT_A, MID), lambda i: (i, 0)),
        out_shape=jax.ShapeDtypeStruct((BATCH, MID), jnp.float32),
    )(x, W1, b1r)

    h2t, cs = pl.pallas_call(
        _enc2_kernel,
        grid=(CODE // CT_B, BATCH // BT_B),
        in_specs=[
            pl.BlockSpec((BT_B, MID), lambda j, i: (i, 0)),
            pl.BlockSpec((MID, CT_B), lambda j, i: (0, j)),
            pl.BlockSpec((1, CT_B), lambda j, i: (0, j)),
        ],
        out_specs=[
            pl.BlockSpec((CT_B, BT_B), lambda j, i: (j, i)),
            pl.BlockSpec((1, CT_B), lambda j, i: (0, j)),
        ],
        out_shape=[
            jax.ShapeDtypeStruct((CODE, BATCH), jnp.float32),
            jax.ShapeDtypeStruct((1, CODE), jnp.float32),
        ],
    )(h1, W2, b2r)

    nmask, slotm = pl.pallas_call(
        _nmask_kernel,
        out_shape=[
            jax.ShapeDtypeStruct((1, CODE), jnp.float32),
            jax.ShapeDtypeStruct((1, CODE), jnp.float32),
        ],
    )(cs)

    idxf, w3ct, selp, selt = pl.pallas_call(
        _build_kernel,
        grid=(CODE // KB,),
        in_specs=[
            pl.BlockSpec((1, KB), lambda k: (0, k)),
            pl.BlockSpec((KB, MID), lambda k: (k, 0)),
        ],
        out_specs=[
            pl.BlockSpec((1, CC), lambda k: (0, 0)),
            pl.BlockSpec((MID, CC), lambda k: (0, 0)),
            pl.BlockSpec((NS, CC), lambda k: (0, 0)),
            pl.BlockSpec((CC, NS), lambda k: (0, 0)),
        ],
        out_shape=[
            jax.ShapeDtypeStruct((1, CC), jnp.float32),
            jax.ShapeDtypeStruct((MID, CC), jnp.bfloat16),
            jax.ShapeDtypeStruct((NS, CC), jnp.bfloat16),
            jax.ShapeDtypeStruct((CC, NS), jnp.bfloat16),
        ],
        scratch_shapes=[
            pltpu.VMEM((CC, MID), jnp.float32),
            pltpu.VMEM((1, CC), jnp.float32),
        ],
    )(slotm, W3)

    idx = idxf.reshape(CC).astype(jnp.int32)

    mesh = plsc.VectorSubcoreMesh(core_axis_name="c", subcore_axis_name="s")
    gather = functools.partial(
        pl.kernel, mesh=mesh,
        out_type=jax.ShapeDtypeStruct((CC, BATCH), jnp.float32),
        scratch_types=[
            pltpu.VMEM((RPW,), jnp.int32),
            pltpu.VMEM((RPW, BATCH), jnp.float32),
            pltpu.SemaphoreType.DMA,
        ],
    )(_sc_gather)
    c = gather(h2t, idx)

    w4t = pl.pallas_call(
        _w4t_kernel,
        in_specs=[pl.BlockSpec((MID, IN_DIM), lambda: (0, 0))],
        out_specs=pl.BlockSpec((IN_DIM, MID), lambda: (0, 0)),
        out_shape=jax.ShapeDtypeStruct((IN_DIM, MID), jnp.bfloat16),
    )(W4)

    out = pl.pallas_call(
        _dect_kernel,
        grid=(BATCH // BT_D,),
        in_specs=[
            pl.BlockSpec((CC, BT_D), lambda i: (0, i)),
            pl.BlockSpec((NS, CC), lambda i: (0, 0)),
            pl.BlockSpec((CC, NS), lambda i: (0, 0)),
            pl.BlockSpec((MID, CC), lambda i: (0, 0)),
            pl.BlockSpec((MID, 1), lambda i: (0, 0)),
            pl.BlockSpec((IN_DIM, MID), lambda i: (0, 0)),
            pl.BlockSpec((IN_DIM, 1), lambda i: (0, 0)),
        ],
        out_specs=pl.BlockSpec((BT_D, IN_DIM), lambda i: (i, 0)),
        out_shape=jax.ShapeDtypeStruct((BATCH, IN_DIM), jnp.float32),
    )(c, selp, selt, w3ct, b3c, w4t, b4c)
    return out
